# baseline (speedup 1.0000x reference)
import functools
import math

import jax
import jax.numpy as jnp
from jax import lax
from jax.experimental import pallas as pl
from jax.experimental.pallas import tpu as pltpu
from jax.experimental.pallas import tpu_sc as plsc

N = 10000
E = 320000
D = 128
NC = 2
NS = 16
NW = NC * NS
EPW = E // NW
CHUNK = 80
NCHUNK = EPW // CHUNK
RPT = 632
NPAD = NS * RPT
DEGW = 8

_BN_SCALE = 1.0 / math.sqrt(1.0 + 1e-5)


def _edge_pass_body(with_deg, *refs):
    if with_deg:
        (y_hbm, src_hbm, dst_hbm, zsum_hbm, zdeg_hbm, ones_hbm,
         sum_out, deg_out,
         acc_sh, deg_sh, src_v, dst_v, rows0, rows1, ones_v,
         sem0, sem1, dsem0, dsem1) = refs
    else:
        (y_hbm, src_hbm, dst_hbm, zsum_hbm,
         sum_out,
         acc_sh, src_v, dst_v, rows0, rows1, sem0, sem1) = refs

    c = lax.axis_index("c")
    s = lax.axis_index("s")
    wid = s * NC + c

    r = pl.ds(s * RPT, RPT)
    pltpu.sync_copy(zsum_hbm.at[r], acc_sh.at[r])
    if with_deg:
        pltpu.sync_copy(zdeg_hbm.at[r], deg_sh.at[r])
        pltpu.sync_copy(ones_hbm, ones_v)

    pltpu.sync_copy(src_hbm.at[wid], src_v)
    pltpu.sync_copy(dst_hbm.at[wid], dst_v)
    plsc.subcore_barrier()

    rows = (rows0, rows1)
    sems = (sem0, sem1)
    if with_deg:
        dsems = (dsem0, dsem1)
    for b in range(2):
        pltpu.async_copy(y_hbm.at[src_v.at[b]], rows[b], sems[b])

    def body(it, carry):
        j = it * 2
        for b in range(2):
            jb = j + b
            pltpu.make_async_copy(y_hbm.at[src_v.at[jb]], rows[b],
                                  sems[b]).wait()
            pltpu.sync_copy(rows[b], acc_sh.at[dst_v.at[jb]], add=True)
            if with_deg:
                @pl.when(jb >= 2)
                def _():
                    pltpu.make_async_copy(
                        ones_v, deg_sh.at[dst_v.at[jb]], dsems[b]).wait()

                pltpu.async_copy(ones_v, deg_sh.at[dst_v.at[jb]], dsems[b],
                                 add=True)
            nxt = jb + 2

            @pl.when(nxt < NCHUNK)
            def _():
                pltpu.async_copy(y_hbm.at[src_v.at[nxt]], rows[b], sems[b])
        return carry

    lax.fori_loop(0, NCHUNK // 2, body, 0)

    if NCHUNK % 2:
        jt = NCHUNK - 1
        bt = jt % 2
        pltpu.make_async_copy(y_hbm.at[src_v.at[jt]], rows[bt],
                              sems[bt]).wait()
        pltpu.sync_copy(rows[bt], acc_sh.at[dst_v.at[jt]], add=True)
        if with_deg:
            pltpu.make_async_copy(ones_v, deg_sh.at[dst_v.at[jt]],
                                  dsems[bt]).wait()
            pltpu.async_copy(ones_v, deg_sh.at[dst_v.at[jt]], dsems[bt],
                             add=True)

    if with_deg:
        for b in range(2):
            pltpu.make_async_copy(ones_v, deg_sh.at[dst_v.at[b]],
                                  dsems[b]).wait()

    plsc.subcore_barrier()
    pltpu.sync_copy(acc_sh.at[r], sum_out.at[c, r])
    if with_deg:
        pltpu.sync_copy(deg_sh.at[r], deg_out.at[c, r])


_SC_MESH = plsc.VectorSubcoreMesh(core_axis_name="c", subcore_axis_name="s")
_SC_PARAMS = pltpu.CompilerParams(use_tc_tiling_on_sc=False)

_edge_pass_deg = pl.kernel(
    functools.partial(_edge_pass_body, True),
    out_type=[jax.ShapeDtypeStruct((NC, NPAD, D), jnp.float32),
              jax.ShapeDtypeStruct((NC, NPAD, DEGW), jnp.float32)],
    mesh=_SC_MESH,
    compiler_params=_SC_PARAMS,
    scratch_types=[
        pltpu.VMEM_SHARED((NPAD, D), jnp.float32),
        pltpu.VMEM_SHARED((NPAD, DEGW), jnp.float32),
        pltpu.VMEM((NCHUNK, CHUNK), jnp.int32),
        pltpu.VMEM((NCHUNK, CHUNK), jnp.int32),
        pltpu.VMEM((CHUNK, D), jnp.float32),
        pltpu.VMEM((CHUNK, D), jnp.float32),
        pltpu.VMEM((CHUNK, DEGW), jnp.float32),
        pltpu.SemaphoreType.DMA,
        pltpu.SemaphoreType.DMA,
        pltpu.SemaphoreType.DMA,
        pltpu.SemaphoreType.DMA,
    ],
)

_edge_pass = pl.kernel(
    functools.partial(_edge_pass_body, False),
    out_type=[jax.ShapeDtypeStruct((NC, NPAD, D), jnp.float32)],
    mesh=_SC_MESH,
    compiler_params=_SC_PARAMS,
    scratch_types=[
        pltpu.VMEM_SHARED((NPAD, D), jnp.float32),
        pltpu.VMEM((NCHUNK, CHUNK), jnp.int32),
        pltpu.VMEM((NCHUNK, CHUNK), jnp.int32),
        pltpu.VMEM((CHUNK, D), jnp.float32),
        pltpu.VMEM((CHUNK, D), jnp.float32),
        pltpu.SemaphoreType.DMA,
        pltpu.SemaphoreType.DMA,
    ],
)


BLK = 1000


def _pre_body(x_ref, w_ref, b_ref, o_ref):
    o_ref[...] = jnp.maximum(
        jnp.dot(x_ref[...], w_ref[...], preferred_element_type=jnp.float32)
        + b_ref[...], 0.0)


_pre = pl.pallas_call(
    _pre_body,
    grid=(N // BLK,),
    in_specs=[pl.BlockSpec((BLK, D), lambda i: (i, 0)),
              pl.BlockSpec((D, D), lambda i: (0, 0)),
              pl.BlockSpec((1, D), lambda i: (0, 0))],
    out_specs=pl.BlockSpec((BLK, D), lambda i: (i, 0)),
    out_shape=jax.ShapeDtypeStruct((N, D), jnp.float32),
)


def _agg_from_parts(s_ref, d_ref):
    summed = s_ref[0] + s_ref[1]
    deg = d_ref[0, :, 0:1] + d_ref[1, :, 0:1]
    return summed * (1.0 / jnp.maximum(deg, 1.0))


def _combine_body(y_ref, s_ref, d_ref, ws_ref, wn_ref, b_ref, g_ref, be_ref,
                  o_ref):
    agg = _agg_from_parts(s_ref, d_ref)
    h = (jnp.dot(y_ref[...], ws_ref[...], preferred_element_type=jnp.float32)
         + jnp.dot(agg, wn_ref[...], preferred_element_type=jnp.float32)
         + b_ref[...])
    h = h * (_BN_SCALE * g_ref[...]) + be_ref[...]
    o_ref[...] = jnp.maximum(h, 0.0)


_combine = pl.pallas_call(
    _combine_body,
    grid=(N // BLK,),
    in_specs=[pl.BlockSpec((BLK, D), lambda i: (i, 0)),
              pl.BlockSpec((NC, BLK, D), lambda i: (0, i, 0)),
              pl.BlockSpec((NC, BLK, DEGW), lambda i: (0, i, 0)),
              pl.BlockSpec((D, D), lambda i: (0, 0)),
              pl.BlockSpec((D, D), lambda i: (0, 0)),
              pl.BlockSpec((1, D), lambda i: (0, 0)),
              pl.BlockSpec((1, D), lambda i: (0, 0)),
              pl.BlockSpec((1, D), lambda i: (0, 0))],
    out_specs=pl.BlockSpec((BLK, D), lambda i: (i, 0)),
    out_shape=jax.ShapeDtypeStruct((N, D), jnp.float32),
)


def _final_body(y_ref, s_ref, d_ref, ws_ref, wn_ref, b_ref, g_ref, be_ref,
                w2_ref, b2_ref, o_ref):
    agg = _agg_from_parts(s_ref, d_ref)
    h = (jnp.dot(y_ref[...], ws_ref[...], preferred_element_type=jnp.float32)
         + jnp.dot(agg, wn_ref[...], preferred_element_type=jnp.float32)
         + b_ref[...])
    h = h * (_BN_SCALE * g_ref[...]) + be_ref[...]
    o_ref[...] = (jnp.dot(h, w2_ref[...], preferred_element_type=jnp.float32)
                  + b2_ref[...])


_final = pl.pallas_call(
    _final_body,
    grid=(N // BLK,),
    in_specs=[pl.BlockSpec((BLK, D), lambda i: (i, 0)),
              pl.BlockSpec((NC, BLK, D), lambda i: (0, i, 0)),
              pl.BlockSpec((NC, BLK, DEGW), lambda i: (0, i, 0)),
              pl.BlockSpec((D, D), lambda i: (0, 0)),
              pl.BlockSpec((D, D), lambda i: (0, 0)),
              pl.BlockSpec((1, D), lambda i: (0, 0)),
              pl.BlockSpec((1, D), lambda i: (0, 0)),
              pl.BlockSpec((1, D), lambda i: (0, 0)),
              pl.BlockSpec((D, 1), lambda i: (0, 0)),
              pl.BlockSpec((1, 1), lambda i: (0, 0))],
    out_specs=pl.BlockSpec((BLK, 1), lambda i: (i, 0)),
    out_shape=jax.ShapeDtypeStruct((N, 1), jnp.float32),
)


def kernel(x, edge_index, W1, b1, Wself0, Wneigh0, bconv0, gamma0, beta0,
           Wself1, Wneigh1, bconv1, gamma1, beta1, W2, b2):
    src3 = edge_index[0].reshape(NW, NCHUNK, CHUNK)
    dst3 = edge_index[1].reshape(NW, NCHUNK, CHUNK)
    zsum = jnp.zeros((NPAD, D), jnp.float32)
    zdeg = jnp.zeros((NPAD, DEGW), jnp.float32)
    ones = jnp.ones((CHUNK, DEGW), jnp.float32)

    y0 = _pre(x, W1, b1.reshape(1, D))
    s0, deg = _edge_pass_deg(y0, src3, dst3, zsum, zdeg, ones)
    y1 = _combine(y0, s0, deg, Wself0, Wneigh0, bconv0.reshape(1, D),
                  gamma0.reshape(1, D), beta0.reshape(1, D))
    (s1,) = _edge_pass(y1, src3, dst3, zsum)
    out = _final(y1, s1, deg, Wself1, Wneigh1, bconv1.reshape(1, D),
                 gamma1.reshape(1, D), beta1.reshape(1, D), W2,
                 b2.reshape(1, 1))
    return out


# speedup vs baseline: 1.1514x; 1.1514x over previous
import math

import jax
import jax.numpy as jnp
from jax import lax
from jax.experimental import pallas as pl
from jax.experimental.pallas import tpu as pltpu
from jax.experimental.pallas import tpu_sc as plsc

N = 10000
E = 320000
D = 128
NC = 2
NS = 16
NW = NC * NS
EPW = E // NW
CHUNK = 100
NCHUNK = EPW // CHUNK
NBUF = 3
RPT = 632
NPAD = NS * RPT
DEGW = 8

_BN_SCALE = 1.0 / math.sqrt(1.0 + 1e-5)

_SC_MESH = plsc.VectorSubcoreMesh(core_axis_name="c", subcore_axis_name="s")
_SC_PARAMS = pltpu.CompilerParams(use_tc_tiling_on_sc=False)


def _deg_pass_body(dst_hbm, zdeg_hbm, ones_hbm, deg_out,
                   deg_sh, dst_v, ones_v, sem0, sem1):
    c = lax.axis_index("c")
    s = lax.axis_index("s")
    wid = s * NC + c

    r = pl.ds(s * RPT, RPT)
    pltpu.sync_copy(zdeg_hbm.at[r], deg_sh.at[r])
    pltpu.sync_copy(dst_hbm.at[wid], dst_v)
    pltpu.sync_copy(ones_hbm, ones_v)
    plsc.subcore_barrier()

    sems = (sem0, sem1)
    for b in range(2):
        pltpu.async_copy(ones_v, deg_sh.at[dst_v.at[b]], sems[b], add=True)

    def body(it, carry):
        j = it * 2
        for b in range(2):
            jb = j + b
            pltpu.make_async_copy(ones_v, deg_sh.at[dst_v.at[jb]],
                                  sems[b]).wait()
            nxt = jb + 2

            @pl.when(nxt < NCHUNK)
            def _():
                pltpu.async_copy(ones_v, deg_sh.at[dst_v.at[nxt]], sems[b],
                                 add=True)
        return carry

    lax.fori_loop(0, NCHUNK // 2, body, 0)

    plsc.subcore_barrier()
    pltpu.sync_copy(deg_sh.at[r], deg_out.at[c, r])


_deg_pass = pl.kernel(
    _deg_pass_body,
    out_type=[jax.ShapeDtypeStruct((NC, NPAD, DEGW), jnp.float32)],
    mesh=_SC_MESH,
    compiler_params=_SC_PARAMS,
    scratch_types=[
        pltpu.VMEM_SHARED((NPAD, DEGW), jnp.float32),
        pltpu.VMEM((NCHUNK, CHUNK), jnp.int32),
        pltpu.VMEM((CHUNK, DEGW), jnp.float32),
        pltpu.SemaphoreType.DMA,
        pltpu.SemaphoreType.DMA,
    ],
)


def _edge_pass_body(y_hbm, src_hbm, dst_hbm, zsum_hbm,
                    sum_out,
                    acc_sh, src_v, dst_c, rows0, rows1, rows2,
                    gsem0, gsem1, gsem2, ssem0, ssem1, ssem2,
                    isem0, isem1, isem2):
    c = lax.axis_index("c")
    s = lax.axis_index("s")
    wid = s * NC + c

    r = pl.ds(s * RPT, RPT)
    pltpu.sync_copy(zsum_hbm.at[r], acc_sh.at[r])

    pltpu.sync_copy(src_hbm.at[wid], src_v)
    plsc.subcore_barrier()

    rows = (rows0, rows1, rows2)
    gsems = (gsem0, gsem1, gsem2)
    ssems = (ssem0, ssem1, ssem2)
    isems = (isem0, isem1, isem2)

    for b in range(2):
        pltpu.async_copy(dst_hbm.at[wid, b], dst_c.at[b], isems[b])
        pltpu.async_copy(y_hbm.at[src_v.at[b]], rows[b], gsems[b])

    def process(jb, b):
        pltpu.make_async_copy(dst_hbm.at[wid, jb], dst_c.at[b],
                              isems[b]).wait()
        pltpu.make_async_copy(y_hbm.at[src_v.at[jb]], rows[b],
                              gsems[b]).wait()
        pltpu.async_copy(rows[b], acc_sh.at[dst_c.at[b]], ssems[b], add=True)

    def refill(jb, b):
        jn = jb + 2
        bn = (b + 2) % NBUF

        @pl.when(jn < NCHUNK)
        def _():
            @pl.when(jn >= NBUF)
            def _():
                pltpu.make_async_copy(rows[bn], acc_sh.at[dst_c.at[bn]],
                                      ssems[bn]).wait()

            pltpu.async_copy(dst_hbm.at[wid, jn], dst_c.at[bn], isems[bn])
            pltpu.async_copy(y_hbm.at[src_v.at[jn]], rows[bn], gsems[bn])

    def body(it, carry):
        j = it * NBUF
        for b in range(NBUF):
            jb = j + b
            process(jb, b)
            refill(jb, b)
        return carry

    lax.fori_loop(0, NCHUNK // NBUF, body, 0)

    for t in range(NCHUNK % NBUF):
        jt = (NCHUNK // NBUF) * NBUF + t
        process(jt, jt % NBUF)

    for b in range(NBUF):
        pltpu.make_async_copy(rows[b], acc_sh.at[dst_c.at[b]],
                              ssems[b]).wait()

    plsc.subcore_barrier()
    pltpu.sync_copy(acc_sh.at[r], sum_out.at[c, r])


_edge_pass = pl.kernel(
    _edge_pass_body,
    out_type=[jax.ShapeDtypeStruct((NC, NPAD, D), jnp.float32)],
    mesh=_SC_MESH,
    compiler_params=_SC_PARAMS,
    scratch_types=[
        pltpu.VMEM_SHARED((NPAD, D), jnp.float32),
        pltpu.VMEM((NCHUNK, CHUNK), jnp.int32),
        pltpu.VMEM((NBUF, CHUNK), jnp.int32),
        pltpu.VMEM((CHUNK, D), jnp.float32),
        pltpu.VMEM((CHUNK, D), jnp.float32),
        pltpu.VMEM((CHUNK, D), jnp.float32),
        pltpu.SemaphoreType.DMA,
        pltpu.SemaphoreType.DMA,
        pltpu.SemaphoreType.DMA,
        pltpu.SemaphoreType.DMA,
        pltpu.SemaphoreType.DMA,
        pltpu.SemaphoreType.DMA,
        pltpu.SemaphoreType.DMA,
        pltpu.SemaphoreType.DMA,
        pltpu.SemaphoreType.DMA,
    ],
)


BLK = 1000


def _pre_body(x_ref, w_ref, b_ref, o_ref):
    o_ref[...] = jnp.maximum(
        jnp.dot(x_ref[...], w_ref[...], preferred_element_type=jnp.float32)
        + b_ref[...], 0.0)


_pre = pl.pallas_call(
    _pre_body,
    grid=(N // BLK,),
    in_specs=[pl.BlockSpec((BLK, D), lambda i: (i, 0)),
              pl.BlockSpec((D, D), lambda i: (0, 0)),
              pl.BlockSpec((1, D), lambda i: (0, 0))],
    out_specs=pl.BlockSpec((BLK, D), lambda i: (i, 0)),
    out_shape=jax.ShapeDtypeStruct((N, D), jnp.float32),
)


def _agg_from_parts(s_ref, d_ref):
    summed = s_ref[0] + s_ref[1]
    deg = d_ref[0, :, 0:1] + d_ref[1, :, 0:1]
    return summed * (1.0 / jnp.maximum(deg, 1.0))


def _combine_body(y_ref, s_ref, d_ref, ws_ref, wn_ref, b_ref, g_ref, be_ref,
                  o_ref):
    agg = _agg_from_parts(s_ref, d_ref)
    h = (jnp.dot(y_ref[...], ws_ref[...], preferred_element_type=jnp.float32)
         + jnp.dot(agg, wn_ref[...], preferred_element_type=jnp.float32)
         + b_ref[...])
    h = h * (_BN_SCALE * g_ref[...]) + be_ref[...]
    o_ref[...] = jnp.maximum(h, 0.0)


_combine = pl.pallas_call(
    _combine_body,
    grid=(N // BLK,),
    in_specs=[pl.BlockSpec((BLK, D), lambda i: (i, 0)),
              pl.BlockSpec((NC, BLK, D), lambda i: (0, i, 0)),
              pl.BlockSpec((NC, BLK, DEGW), lambda i: (0, i, 0)),
              pl.BlockSpec((D, D), lambda i: (0, 0)),
              pl.BlockSpec((D, D), lambda i: (0, 0)),
              pl.BlockSpec((1, D), lambda i: (0, 0)),
              pl.BlockSpec((1, D), lambda i: (0, 0)),
              pl.BlockSpec((1, D), lambda i: (0, 0))],
    out_specs=pl.BlockSpec((BLK, D), lambda i: (i, 0)),
    out_shape=jax.ShapeDtypeStruct((N, D), jnp.float32),
)


def _final_body(y_ref, s_ref, d_ref, ws_ref, wn_ref, b_ref, g_ref, be_ref,
                w2_ref, b2_ref, o_ref):
    agg = _agg_from_parts(s_ref, d_ref)
    h = (jnp.dot(y_ref[...], ws_ref[...], preferred_element_type=jnp.float32)
         + jnp.dot(agg, wn_ref[...], preferred_element_type=jnp.float32)
         + b_ref[...])
    h = h * (_BN_SCALE * g_ref[...]) + be_ref[...]
    o_ref[...] = (jnp.dot(h, w2_ref[...], preferred_element_type=jnp.float32)
                  + b2_ref[...])


_final = pl.pallas_call(
    _final_body,
    grid=(N // BLK,),
    in_specs=[pl.BlockSpec((BLK, D), lambda i: (i, 0)),
              pl.BlockSpec((NC, BLK, D), lambda i: (0, i, 0)),
              pl.BlockSpec((NC, BLK, DEGW), lambda i: (0, i, 0)),
              pl.BlockSpec((D, D), lambda i: (0, 0)),
              pl.BlockSpec((D, D), lambda i: (0, 0)),
              pl.BlockSpec((1, D), lambda i: (0, 0)),
              pl.BlockSpec((1, D), lambda i: (0, 0)),
              pl.BlockSpec((1, D), lambda i: (0, 0)),
              pl.BlockSpec((D, 1), lambda i: (0, 0)),
              pl.BlockSpec((1, 1), lambda i: (0, 0))],
    out_specs=pl.BlockSpec((BLK, 1), lambda i: (i, 0)),
    out_shape=jax.ShapeDtypeStruct((N, 1), jnp.float32),
)


def kernel(x, edge_index, W1, b1, Wself0, Wneigh0, bconv0, gamma0, beta0,
           Wself1, Wneigh1, bconv1, gamma1, beta1, W2, b2):
    src3 = edge_index[0].reshape(NW, NCHUNK, CHUNK)
    dst3 = edge_index[1].reshape(NW, NCHUNK, CHUNK)
    zsum = jnp.zeros((NPAD, D), jnp.float32)
    zdeg = jnp.zeros((NPAD, DEGW), jnp.float32)
    ones = jnp.ones((CHUNK, DEGW), jnp.float32)

    (deg,) = _deg_pass(dst3, zdeg, ones)
    y0 = _pre(x, W1, b1.reshape(1, D))
    (s0,) = _edge_pass(y0, src3, dst3, zsum)
    y1 = _combine(y0, s0, deg, Wself0, Wneigh0, bconv0.reshape(1, D),
                  gamma0.reshape(1, D), beta0.reshape(1, D))
    (s1,) = _edge_pass(y1, src3, dst3, zsum)
    out = _final(y1, s1, deg, Wself1, Wneigh1, bconv1.reshape(1, D),
                 gamma1.reshape(1, D), beta1.reshape(1, D), W2,
                 b2.reshape(1, 1))
    return out


# speedup vs baseline: 1.1679x; 1.0143x over previous
import math

import jax
import jax.numpy as jnp
from jax import lax
from jax.experimental import pallas as pl
from jax.experimental.pallas import tpu as pltpu
from jax.experimental.pallas import tpu_sc as plsc

N = 10000
E = 320000
D = 128
NC = 2
NS = 16
NW = NC * NS
EPW = E // NW
CHUNK = 80
NCHUNK = EPW // CHUNK
NBUF = 3
NBUFD = 6
DLEAD = 3
RPT = 632
NPAD = NS * RPT
DEGW = 8
ZROWS = 40

_BN_SCALE = 1.0 / math.sqrt(1.0 + 1e-5)

_SC_MESH = plsc.VectorSubcoreMesh(core_axis_name="c", subcore_axis_name="s")
_SC_PARAMS = pltpu.CompilerParams(use_tc_tiling_on_sc=False)


def _dst_slice(ei_hbm, wid, j):
    off = pl.multiple_of(wid * EPW + j * CHUNK, 8)
    return ei_hbm.at[1, pl.ds(off, CHUNK)]


def _deg_pass_body(ei_hbm, zdeg_hbm, ones_hbm, deg_out,
                   deg_sh, dst_c, ones_v, *sems):
    isems = sems[:NBUFD]
    dsems = sems[NBUFD:]
    c = lax.axis_index("c")
    s = lax.axis_index("s")
    wid = s * NC + c

    r = pl.ds(s * RPT, RPT)
    pltpu.sync_copy(zdeg_hbm.at[r], deg_sh.at[r])
    pltpu.sync_copy(ones_hbm, ones_v)
    plsc.subcore_barrier()

    for b in range(DLEAD):
        pltpu.async_copy(_dst_slice(ei_hbm, wid, b), dst_c.at[b], isems[b])

    def process(jb, b):
        pltpu.make_async_copy(_dst_slice(ei_hbm, wid, jb), dst_c.at[b],
                              isems[b]).wait()
        pltpu.async_copy(ones_v, deg_sh.at[dst_c.at[b]], dsems[b], add=True)

    def refill(jb, b):
        jn = jb + DLEAD
        bn = (b + DLEAD) % NBUFD

        @pl.when(jn < NCHUNK)
        def _():
            @pl.when(jn >= NBUFD)
            def _():
                pltpu.make_async_copy(ones_v, deg_sh.at[dst_c.at[bn]],
                                      dsems[bn]).wait()

            pltpu.async_copy(_dst_slice(ei_hbm, wid, jn), dst_c.at[bn],
                             isems[bn])

    def body(it, carry):
        j = it * NBUFD
        for b in range(NBUFD):
            process(j + b, b)
            refill(j + b, b)
        return carry

    lax.fori_loop(0, NCHUNK // NBUFD, body, 0)

    for t in range(NCHUNK % NBUFD):
        jt = (NCHUNK // NBUFD) * NBUFD + t
        process(jt, jt % NBUFD)
        refill(jt, jt % NBUFD)

    for b in range(NBUFD):
        pltpu.make_async_copy(ones_v, deg_sh.at[dst_c.at[b]],
                              dsems[b]).wait()

    plsc.subcore_barrier()
    pltpu.sync_copy(deg_sh.at[r], deg_out.at[c, r])


_deg_pass = pl.kernel(
    _deg_pass_body,
    out_type=[jax.ShapeDtypeStruct((NC, NPAD, DEGW), jnp.float32)],
    mesh=_SC_MESH,
    compiler_params=_SC_PARAMS,
    scratch_types=[
        pltpu.VMEM_SHARED((NPAD, DEGW), jnp.float32),
        pltpu.VMEM((NBUFD, CHUNK), jnp.int32),
        pltpu.VMEM((CHUNK, DEGW), jnp.float32),
    ] + [pltpu.SemaphoreType.DMA] * (2 * NBUFD),
)


def _edge_pass_body(ei_hbm, y_hbm,
                    sum_out,
                    acc_sh, src_v, dst_c, rows0, rows1, rows2, zbuf, *sems):
    gsems = sems[0:NBUF]
    ssems = sems[NBUF:2 * NBUF]
    isems = sems[2 * NBUF:]
    c = lax.axis_index("c")
    s = lax.axis_index("s")
    wid = s * NC + c

    for i in range(ZROWS):
        for k in range(D // 16):
            zbuf[i, pl.ds(k * 16, 16)] = jnp.zeros((16,), jnp.float32)
    base_r = s * RPT
    for i in range(RPT // ZROWS):
        pltpu.sync_copy(zbuf, acc_sh.at[pl.ds(base_r + i * ZROWS, ZROWS)])
    rem = RPT % ZROWS
    if rem:
        pltpu.sync_copy(zbuf.at[pl.ds(0, rem)],
                        acc_sh.at[pl.ds(base_r + (RPT // ZROWS) * ZROWS,
                                        rem)])

    soff = pl.multiple_of(wid * EPW, 8)
    pltpu.sync_copy(ei_hbm.at[0, pl.ds(soff, EPW)], src_v)
    plsc.subcore_barrier()

    rows = (rows0, rows1, rows2)

    def src_idx(j):
        return src_v.at[pl.ds(pl.multiple_of(j * CHUNK, 8), CHUNK)]

    for b in range(2):
        pltpu.async_copy(_dst_slice(ei_hbm, wid, b), dst_c.at[b], isems[b])
        pltpu.async_copy(y_hbm.at[src_idx(b)], rows[b], gsems[b])

    def process(jb, b):
        pltpu.make_async_copy(_dst_slice(ei_hbm, wid, jb), dst_c.at[b],
                              isems[b]).wait()
        pltpu.make_async_copy(y_hbm.at[src_idx(jb)], rows[b],
                              gsems[b]).wait()
        pltpu.async_copy(rows[b], acc_sh.at[dst_c.at[b]], ssems[b], add=True)

    def refill(jb, b):
        jn = jb + 2
        bn = (b + 2) % NBUF

        @pl.when(jn < NCHUNK)
        def _():
            @pl.when(jn >= NBUF)
            def _():
                pltpu.make_async_copy(rows[bn], acc_sh.at[dst_c.at[bn]],
                                      ssems[bn]).wait()

            pltpu.async_copy(_dst_slice(ei_hbm, wid, jn), dst_c.at[bn],
                             isems[bn])
            pltpu.async_copy(y_hbm.at[src_idx(jn)], rows[bn], gsems[bn])

    def body(it, carry):
        j = it * NBUF
        for b in range(NBUF):
            jb = j + b
            process(jb, b)
            refill(jb, b)
        return carry

    lax.fori_loop(0, NCHUNK // NBUF, body, 0)

    for t in range(NCHUNK % NBUF):
        jt = (NCHUNK // NBUF) * NBUF + t
        process(jt, jt % NBUF)

    for b in range(NBUF):
        pltpu.make_async_copy(rows[b], acc_sh.at[dst_c.at[b]],
                              ssems[b]).wait()

    plsc.subcore_barrier()
    r = pl.ds(base_r, RPT)
    pltpu.sync_copy(acc_sh.at[r], sum_out.at[c, r])


_edge_pass = pl.kernel(
    _edge_pass_body,
    out_type=[jax.ShapeDtypeStruct((NC, NPAD, D), jnp.float32)],
    mesh=_SC_MESH,
    compiler_params=_SC_PARAMS,
    scratch_types=[
        pltpu.VMEM_SHARED((NPAD, D), jnp.float32),
        pltpu.VMEM((EPW,), jnp.int32),
        pltpu.VMEM((NBUF, CHUNK), jnp.int32),
        pltpu.VMEM((CHUNK, D), jnp.float32),
        pltpu.VMEM((CHUNK, D), jnp.float32),
        pltpu.VMEM((CHUNK, D), jnp.float32),
        pltpu.VMEM((ZROWS, D), jnp.float32),
    ] + [pltpu.SemaphoreType.DMA] * (3 * NBUF),
)


BLK = 1000


def _pre_body(x_ref, w_ref, b_ref, o_ref):
    o_ref[...] = jnp.maximum(
        jnp.dot(x_ref[...], w_ref[...], preferred_element_type=jnp.float32)
        + b_ref[...], 0.0)


_pre = pl.pallas_call(
    _pre_body,
    grid=(N // BLK,),
    in_specs=[pl.BlockSpec((BLK, D), lambda i: (i, 0)),
              pl.BlockSpec((D, D), lambda i: (0, 0)),
              pl.BlockSpec((1, D), lambda i: (0, 0))],
    out_specs=pl.BlockSpec((BLK, D), lambda i: (i, 0)),
    out_shape=jax.ShapeDtypeStruct((N, D), jnp.float32),
)


def _agg_from_parts(s_ref, d_ref):
    summed = s_ref[0] + s_ref[1]
    deg = d_ref[0, :, 0:1] + d_ref[1, :, 0:1]
    return summed * (1.0 / jnp.maximum(deg, 1.0))


def _combine_body(y_ref, s_ref, d_ref, ws_ref, wn_ref, b_ref, g_ref, be_ref,
                  o_ref):
    agg = _agg_from_parts(s_ref, d_ref)
    h = (jnp.dot(y_ref[...], ws_ref[...], preferred_element_type=jnp.float32)
         + jnp.dot(agg, wn_ref[...], preferred_element_type=jnp.float32)
         + b_ref[...])
    h = h * (_BN_SCALE * g_ref[...]) + be_ref[...]
    o_ref[...] = jnp.maximum(h, 0.0)


_combine = pl.pallas_call(
    _combine_body,
    grid=(N // BLK,),
    in_specs=[pl.BlockSpec((BLK, D), lambda i: (i, 0)),
              pl.BlockSpec((NC, BLK, D), lambda i: (0, i, 0)),
              pl.BlockSpec((NC, BLK, DEGW), lambda i: (0, i, 0)),
              pl.BlockSpec((D, D), lambda i: (0, 0)),
              pl.BlockSpec((D, D), lambda i: (0, 0)),
              pl.BlockSpec((1, D), lambda i: (0, 0)),
              pl.BlockSpec((1, D), lambda i: (0, 0)),
              pl.BlockSpec((1, D), lambda i: (0, 0))],
    out_specs=pl.BlockSpec((BLK, D), lambda i: (i, 0)),
    out_shape=jax.ShapeDtypeStruct((N, D), jnp.float32),
)


def _final_body(y_ref, s_ref, d_ref, ws_ref, wn_ref, b_ref, g_ref, be_ref,
                w2_ref, b2_ref, o_ref):
    agg = _agg_from_parts(s_ref, d_ref)
    h = (jnp.dot(y_ref[...], ws_ref[...], preferred_element_type=jnp.float32)
         + jnp.dot(agg, wn_ref[...], preferred_element_type=jnp.float32)
         + b_ref[...])
    h = h * (_BN_SCALE * g_ref[...]) + be_ref[...]
    o_ref[...] = (jnp.dot(h, w2_ref[...], preferred_element_type=jnp.float32)
                  + b2_ref[...])


_final = pl.pallas_call(
    _final_body,
    grid=(N // BLK,),
    in_specs=[pl.BlockSpec((BLK, D), lambda i: (i, 0)),
              pl.BlockSpec((NC, BLK, D), lambda i: (0, i, 0)),
              pl.BlockSpec((NC, BLK, DEGW), lambda i: (0, i, 0)),
              pl.BlockSpec((D, D), lambda i: (0, 0)),
              pl.BlockSpec((D, D), lambda i: (0, 0)),
              pl.BlockSpec((1, D), lambda i: (0, 0)),
              pl.BlockSpec((1, D), lambda i: (0, 0)),
              pl.BlockSpec((1, D), lambda i: (0, 0)),
              pl.BlockSpec((D, 1), lambda i: (0, 0)),
              pl.BlockSpec((1, 1), lambda i: (0, 0))],
    out_specs=pl.BlockSpec((BLK, 1), lambda i: (i, 0)),
    out_shape=jax.ShapeDtypeStruct((N, 1), jnp.float32),
)


def kernel(x, edge_index, W1, b1, Wself0, Wneigh0, bconv0, gamma0, beta0,
           Wself1, Wneigh1, bconv1, gamma1, beta1, W2, b2):
    zdeg = jnp.zeros((NPAD, DEGW), jnp.float32)
    ones = jnp.ones((CHUNK, DEGW), jnp.float32)

    (deg,) = _deg_pass(edge_index, zdeg, ones)
    y0 = _pre(x, W1, b1.reshape(1, D))
    (s0,) = _edge_pass(edge_index, y0)
    y1 = _combine(y0, s0, deg, Wself0, Wneigh0, bconv0.reshape(1, D),
                  gamma0.reshape(1, D), beta0.reshape(1, D))
    (s1,) = _edge_pass(edge_index, y1)
    out = _final(y1, s1, deg, Wself1, Wneigh1, bconv1.reshape(1, D),
                 gamma1.reshape(1, D), beta1.reshape(1, D), W2,
                 b2.reshape(1, 1))
    return out


# speedup vs baseline: 1.2191x; 1.0439x over previous
import math

import jax
import jax.numpy as jnp
from jax import lax
from jax.experimental import pallas as pl
from jax.experimental.pallas import tpu as pltpu
from jax.experimental.pallas import tpu_sc as plsc

N = 10000
E = 320000
D = 128
NC = 2
NS = 16
NW = NC * NS
EPW = E // NW
CHUNK = 80
NCHUNK = EPW // CHUNK
NBUF = 3
NBUFD = 6
DLEAD = 3
RPT = 632
NPAD = NS * RPT
DEGW = 8
ZROWS = 40

_BN_SCALE = 1.0 / math.sqrt(1.0 + 1e-5)

_SC_MESH = plsc.VectorSubcoreMesh(core_axis_name="c", subcore_axis_name="s")
_SC_PARAMS = pltpu.CompilerParams(use_tc_tiling_on_sc=False)


def _dst_slice(ei_hbm, wid, j):
    off = pl.multiple_of(wid * EPW + j * CHUNK, 8)
    return ei_hbm.at[1, pl.ds(off, CHUNK)]


def _deg_pass_body(ei_hbm, zdeg_hbm, ones_hbm, deg_out,
                   deg_sh, dst_v, ones_v, sem0, sem1):
    c = lax.axis_index("c")
    s = lax.axis_index("s")
    wid = s * NC + c

    r = pl.ds(s * RPT, RPT)
    pltpu.sync_copy(zdeg_hbm.at[r], deg_sh.at[r])
    soff = pl.multiple_of(wid * EPW, 8)
    pltpu.sync_copy(ei_hbm.at[1, pl.ds(soff, EPW)], dst_v)
    pltpu.sync_copy(ones_hbm, ones_v)
    plsc.subcore_barrier()

    def didx(j):
        return dst_v.at[pl.ds(pl.multiple_of(j * CHUNK, 8), CHUNK)]

    sems = (sem0, sem1)
    for b in range(2):
        pltpu.async_copy(ones_v, deg_sh.at[didx(b)], sems[b], add=True)

    def body(it, carry):
        j = it * 2
        for b in range(2):
            jb = j + b
            pltpu.make_async_copy(ones_v, deg_sh.at[didx(jb)],
                                  sems[b]).wait()
            nxt = jb + 2

            @pl.when(nxt < NCHUNK)
            def _():
                pltpu.async_copy(ones_v, deg_sh.at[didx(nxt)], sems[b],
                                 add=True)
        return carry

    lax.fori_loop(0, NCHUNK // 2, body, 0)

    if NCHUNK % 2:
        jt = NCHUNK - 1
        pltpu.make_async_copy(ones_v, deg_sh.at[didx(jt)],
                              sems[jt % 2]).wait()

    plsc.subcore_barrier()
    pltpu.sync_copy(deg_sh.at[r], deg_out.at[c, r])


_deg_pass = pl.kernel(
    _deg_pass_body,
    out_type=[jax.ShapeDtypeStruct((NC, NPAD, DEGW), jnp.float32)],
    mesh=_SC_MESH,
    compiler_params=_SC_PARAMS,
    scratch_types=[
        pltpu.VMEM_SHARED((NPAD, DEGW), jnp.float32),
        pltpu.VMEM((EPW,), jnp.int32),
        pltpu.VMEM((CHUNK, DEGW), jnp.float32),
        pltpu.SemaphoreType.DMA,
        pltpu.SemaphoreType.DMA,
    ],
)


def _edge_pass_body(ei_hbm, y_hbm,
                    sum_out,
                    acc_sh, src_v, dst_c, rows0, rows1, rows2, zbuf, *sems):
    gsems = sems[0:NBUF]
    ssems = sems[NBUF:2 * NBUF]
    isems = sems[2 * NBUF:]
    c = lax.axis_index("c")
    s = lax.axis_index("s")
    wid = s * NC + c

    for i in range(ZROWS):
        for k in range(D // 16):
            zbuf[i, pl.ds(k * 16, 16)] = jnp.zeros((16,), jnp.float32)
    base_r = s * RPT
    for i in range(RPT // ZROWS):
        pltpu.sync_copy(zbuf, acc_sh.at[pl.ds(base_r + i * ZROWS, ZROWS)])
    rem = RPT % ZROWS
    if rem:
        pltpu.sync_copy(zbuf.at[pl.ds(0, rem)],
                        acc_sh.at[pl.ds(base_r + (RPT // ZROWS) * ZROWS,
                                        rem)])

    soff = pl.multiple_of(wid * EPW, 8)
    pltpu.sync_copy(ei_hbm.at[0, pl.ds(soff, EPW)], src_v)
    plsc.subcore_barrier()

    rows = (rows0, rows1, rows2)

    def src_idx(j):
        return src_v.at[pl.ds(pl.multiple_of(j * CHUNK, 8), CHUNK)]

    for b in range(2):
        pltpu.async_copy(_dst_slice(ei_hbm, wid, b), dst_c.at[b], isems[b])
        pltpu.async_copy(y_hbm.at[src_idx(b)], rows[b], gsems[b])

    def process(jb, b):
        pltpu.make_async_copy(_dst_slice(ei_hbm, wid, jb), dst_c.at[b],
                              isems[b]).wait()
        pltpu.make_async_copy(y_hbm.at[src_idx(jb)], rows[b],
                              gsems[b]).wait()
        pltpu.async_copy(rows[b], acc_sh.at[dst_c.at[b]], ssems[b], add=True)

    def refill(jb, b):
        jn = jb + 2
        bn = (b + 2) % NBUF

        @pl.when(jn < NCHUNK)
        def _():
            @pl.when(jn >= NBUF)
            def _():
                pltpu.make_async_copy(rows[bn], acc_sh.at[dst_c.at[bn]],
                                      ssems[bn]).wait()

            pltpu.async_copy(_dst_slice(ei_hbm, wid, jn), dst_c.at[bn],
                             isems[bn])
            pltpu.async_copy(y_hbm.at[src_idx(jn)], rows[bn], gsems[bn])

    def body(it, carry):
        j = it * NBUF
        for b in range(NBUF):
            jb = j + b
            process(jb, b)
            refill(jb, b)
        return carry

    lax.fori_loop(0, NCHUNK // NBUF, body, 0)

    for t in range(NCHUNK % NBUF):
        jt = (NCHUNK // NBUF) * NBUF + t
        process(jt, jt % NBUF)

    for b in range(NBUF):
        pltpu.make_async_copy(rows[b], acc_sh.at[dst_c.at[b]],
                              ssems[b]).wait()

    plsc.subcore_barrier()
    r = pl.ds(base_r, RPT)
    pltpu.sync_copy(acc_sh.at[r], sum_out.at[c, r])


_edge_pass = pl.kernel(
    _edge_pass_body,
    out_type=[jax.ShapeDtypeStruct((NC, NPAD, D), jnp.float32)],
    mesh=_SC_MESH,
    compiler_params=_SC_PARAMS,
    scratch_types=[
        pltpu.VMEM_SHARED((NPAD, D), jnp.float32),
        pltpu.VMEM((EPW,), jnp.int32),
        pltpu.VMEM((NBUF, CHUNK), jnp.int32),
        pltpu.VMEM((CHUNK, D), jnp.float32),
        pltpu.VMEM((CHUNK, D), jnp.float32),
        pltpu.VMEM((CHUNK, D), jnp.float32),
        pltpu.VMEM((ZROWS, D), jnp.float32),
    ] + [pltpu.SemaphoreType.DMA] * (3 * NBUF),
)


BLK = 1000


def _pre_body(x_ref, w_ref, b_ref, o_ref):
    o_ref[...] = jnp.maximum(
        jnp.dot(x_ref[...], w_ref[...], preferred_element_type=jnp.float32)
        + b_ref[...], 0.0)


_pre = pl.pallas_call(
    _pre_body,
    grid=(N // BLK,),
    in_specs=[pl.BlockSpec((BLK, D), lambda i: (i, 0)),
              pl.BlockSpec((D, D), lambda i: (0, 0)),
              pl.BlockSpec((1, D), lambda i: (0, 0))],
    out_specs=pl.BlockSpec((BLK, D), lambda i: (i, 0)),
    out_shape=jax.ShapeDtypeStruct((N, D), jnp.float32),
)


def _agg_from_parts(s_ref, d_ref):
    summed = s_ref[0] + s_ref[1]
    deg = d_ref[0, :, 0:1] + d_ref[1, :, 0:1]
    return summed * (1.0 / jnp.maximum(deg, 1.0))


def _combine_body(y_ref, s_ref, d_ref, ws_ref, wn_ref, b_ref, g_ref, be_ref,
                  o_ref):
    agg = _agg_from_parts(s_ref, d_ref)
    h = (jnp.dot(y_ref[...], ws_ref[...], preferred_element_type=jnp.float32)
         + jnp.dot(agg, wn_ref[...], preferred_element_type=jnp.float32)
         + b_ref[...])
    h = h * (_BN_SCALE * g_ref[...]) + be_ref[...]
    o_ref[...] = jnp.maximum(h, 0.0)


_combine = pl.pallas_call(
    _combine_body,
    grid=(N // BLK,),
    in_specs=[pl.BlockSpec((BLK, D), lambda i: (i, 0)),
              pl.BlockSpec((NC, BLK, D), lambda i: (0, i, 0)),
              pl.BlockSpec((NC, BLK, DEGW), lambda i: (0, i, 0)),
              pl.BlockSpec((D, D), lambda i: (0, 0)),
              pl.BlockSpec((D, D), lambda i: (0, 0)),
              pl.BlockSpec((1, D), lambda i: (0, 0)),
              pl.BlockSpec((1, D), lambda i: (0, 0)),
              pl.BlockSpec((1, D), lambda i: (0, 0))],
    out_specs=pl.BlockSpec((BLK, D), lambda i: (i, 0)),
    out_shape=jax.ShapeDtypeStruct((N, D), jnp.float32),
)


def _final_body(y_ref, s_ref, d_ref, ws_ref, wn_ref, b_ref, g_ref, be_ref,
                w2_ref, b2_ref, o_ref):
    agg = _agg_from_parts(s_ref, d_ref)
    h = (jnp.dot(y_ref[...], ws_ref[...], preferred_element_type=jnp.float32)
         + jnp.dot(agg, wn_ref[...], preferred_element_type=jnp.float32)
         + b_ref[...])
    h = h * (_BN_SCALE * g_ref[...]) + be_ref[...]
    o_ref[...] = (jnp.dot(h, w2_ref[...], preferred_element_type=jnp.float32)
                  + b2_ref[...])


_final = pl.pallas_call(
    _final_body,
    grid=(N // BLK,),
    in_specs=[pl.BlockSpec((BLK, D), lambda i: (i, 0)),
              pl.BlockSpec((NC, BLK, D), lambda i: (0, i, 0)),
              pl.BlockSpec((NC, BLK, DEGW), lambda i: (0, i, 0)),
              pl.BlockSpec((D, D), lambda i: (0, 0)),
              pl.BlockSpec((D, D), lambda i: (0, 0)),
              pl.BlockSpec((1, D), lambda i: (0, 0)),
              pl.BlockSpec((1, D), lambda i: (0, 0)),
              pl.BlockSpec((1, D), lambda i: (0, 0)),
              pl.BlockSpec((D, 1), lambda i: (0, 0)),
              pl.BlockSpec((1, 1), lambda i: (0, 0))],
    out_specs=pl.BlockSpec((BLK, 1), lambda i: (i, 0)),
    out_shape=jax.ShapeDtypeStruct((N, 1), jnp.float32),
)


def kernel(x, edge_index, W1, b1, Wself0, Wneigh0, bconv0, gamma0, beta0,
           Wself1, Wneigh1, bconv1, gamma1, beta1, W2, b2):
    zdeg = jnp.zeros((NPAD, DEGW), jnp.float32)
    ones = jnp.ones((CHUNK, DEGW), jnp.float32)

    (deg,) = _deg_pass(edge_index, zdeg, ones)
    y0 = _pre(x, W1, b1.reshape(1, D))
    (s0,) = _edge_pass(edge_index, y0)
    y1 = _combine(y0, s0, deg, Wself0, Wneigh0, bconv0.reshape(1, D),
                  gamma0.reshape(1, D), beta0.reshape(1, D))
    (s1,) = _edge_pass(edge_index, y1)
    out = _final(y1, s1, deg, Wself1, Wneigh1, bconv1.reshape(1, D),
                 gamma1.reshape(1, D), beta1.reshape(1, D), W2,
                 b2.reshape(1, 1))
    return out


# speedup vs baseline: 1.2198x; 1.0005x over previous
import math

import jax
import jax.numpy as jnp
from jax import lax
from jax.experimental import pallas as pl
from jax.experimental.pallas import tpu as pltpu
from jax.experimental.pallas import tpu_sc as plsc

N = 10000
E = 320000
D = 128
NC = 2
NS = 16
NW = NC * NS
EPW = E // NW
CHUNK = 80
NCHUNK = EPW // CHUNK
NBUF = 3
RPT = 632
NPAD = NS * RPT
DEGW = 8
ZROWS = 40

_BN_SCALE = 1.0 / math.sqrt(1.0 + 1e-5)

_SC_MESH = plsc.VectorSubcoreMesh(core_axis_name="c", subcore_axis_name="s")
_SC_PARAMS = pltpu.CompilerParams(use_tc_tiling_on_sc=False)


def _dst_slice(ei_hbm, wid, j):
    off = pl.multiple_of(wid * EPW + j * CHUNK, 8)
    return ei_hbm.at[1, pl.ds(off, CHUNK)]


def _deg_pass_body(ei_hbm, zdeg_hbm, ones_hbm, deg_out,
                   deg_sh, dst_v, ones_v, sem0, sem1):
    c = lax.axis_index("c")
    s = lax.axis_index("s")
    wid = s * NC + c

    r = pl.ds(s * RPT, RPT)
    pltpu.sync_copy(zdeg_hbm.at[r], deg_sh.at[r])
    soff = pl.multiple_of(wid * EPW, 8)
    pltpu.sync_copy(ei_hbm.at[1, pl.ds(soff, EPW)], dst_v)
    pltpu.sync_copy(ones_hbm, ones_v)
    plsc.subcore_barrier()

    def didx(j):
        return dst_v.at[pl.ds(pl.multiple_of(j * CHUNK, 8), CHUNK)]

    sems = (sem0, sem1)
    for b in range(2):
        pltpu.async_copy(ones_v, deg_sh.at[didx(b)], sems[b], add=True)

    def body(it, carry):
        j = it * 2
        for b in range(2):
            jb = j + b
            pltpu.make_async_copy(ones_v, deg_sh.at[didx(jb)],
                                  sems[b]).wait()
            nxt = jb + 2

            @pl.when(nxt < NCHUNK)
            def _():
                pltpu.async_copy(ones_v, deg_sh.at[didx(nxt)], sems[b],
                                 add=True)
        return carry

    lax.fori_loop(0, NCHUNK // 2, body, 0)

    if NCHUNK % 2:
        jt = NCHUNK - 1
        pltpu.make_async_copy(ones_v, deg_sh.at[didx(jt)],
                              sems[jt % 2]).wait()

    plsc.subcore_barrier()
    pltpu.sync_copy(deg_sh.at[r], deg_out.at[c, r])


_deg_pass = pl.kernel(
    _deg_pass_body,
    out_type=[jax.ShapeDtypeStruct((NC, NPAD, DEGW), jnp.float32)],
    mesh=_SC_MESH,
    compiler_params=_SC_PARAMS,
    scratch_types=[
        pltpu.VMEM_SHARED((NPAD, DEGW), jnp.float32),
        pltpu.VMEM((EPW,), jnp.int32),
        pltpu.VMEM((CHUNK, DEGW), jnp.float32),
        pltpu.SemaphoreType.DMA,
        pltpu.SemaphoreType.DMA,
    ],
)


def _edge_pass_body(ei_hbm, y_hbm,
                    sum_out,
                    acc_sh, src_v, dst_c, rows0, rows1, rows2, zbuf, *sems):
    gsems = sems[0:NBUF]
    ssems = sems[NBUF:2 * NBUF]
    isems = sems[2 * NBUF:]
    c = lax.axis_index("c")
    s = lax.axis_index("s")
    wid = s * NC + c

    for i in range(ZROWS):
        for k in range(D // 16):
            zbuf[i, pl.ds(k * 16, 16)] = jnp.zeros((16,), jnp.float32)
    base_r = s * RPT
    for i in range(RPT // ZROWS):
        pltpu.sync_copy(zbuf, acc_sh.at[pl.ds(base_r + i * ZROWS, ZROWS)])
    rem = RPT % ZROWS
    if rem:
        pltpu.sync_copy(zbuf.at[pl.ds(0, rem)],
                        acc_sh.at[pl.ds(base_r + (RPT // ZROWS) * ZROWS,
                                        rem)])

    soff = pl.multiple_of(wid * EPW, 8)
    pltpu.sync_copy(ei_hbm.at[0, pl.ds(soff, EPW)], src_v)
    plsc.subcore_barrier()

    rows = (rows0, rows1, rows2)

    def src_idx(j):
        return src_v.at[pl.ds(pl.multiple_of(j * CHUNK, 8), CHUNK)]

    for b in range(2):
        pltpu.async_copy(_dst_slice(ei_hbm, wid, b), dst_c.at[b], isems[b])
        pltpu.async_copy(y_hbm.at[src_idx(b)], rows[b], gsems[b])

    def process(jb, b):
        pltpu.make_async_copy(_dst_slice(ei_hbm, wid, jb), dst_c.at[b],
                              isems[b]).wait()
        pltpu.make_async_copy(y_hbm.at[src_idx(jb)], rows[b],
                              gsems[b]).wait()
        pltpu.async_copy(rows[b], acc_sh.at[dst_c.at[b]], ssems[b], add=True)

    def refill(jb, b):
        jn = jb + 2
        bn = (b + 2) % NBUF

        @pl.when(jn < NCHUNK)
        def _():
            @pl.when(jn >= NBUF)
            def _():
                pltpu.make_async_copy(rows[bn], acc_sh.at[dst_c.at[bn]],
                                      ssems[bn]).wait()

            pltpu.async_copy(_dst_slice(ei_hbm, wid, jn), dst_c.at[bn],
                             isems[bn])
            pltpu.async_copy(y_hbm.at[src_idx(jn)], rows[bn], gsems[bn])

    def body(it, carry):
        j = it * NBUF
        for b in range(NBUF):
            jb = j + b
            process(jb, b)
            refill(jb, b)
        return carry

    lax.fori_loop(0, NCHUNK // NBUF, body, 0)

    for t in range(NCHUNK % NBUF):
        jt = (NCHUNK // NBUF) * NBUF + t
        process(jt, jt % NBUF)

    for b in range(NBUF):
        pltpu.make_async_copy(rows[b], acc_sh.at[dst_c.at[b]],
                              ssems[b]).wait()

    plsc.subcore_barrier()
    r = pl.ds(base_r, RPT)
    pltpu.sync_copy(acc_sh.at[r], sum_out.at[c, r])


_edge_pass = pl.kernel(
    _edge_pass_body,
    out_type=[jax.ShapeDtypeStruct((NC, NPAD, D), jnp.float32)],
    mesh=_SC_MESH,
    compiler_params=_SC_PARAMS,
    scratch_types=[
        pltpu.VMEM_SHARED((NPAD, D), jnp.float32),
        pltpu.VMEM((EPW,), jnp.int32),
        pltpu.VMEM((NBUF, CHUNK), jnp.int32),
        pltpu.VMEM((CHUNK, D), jnp.float32),
        pltpu.VMEM((CHUNK, D), jnp.float32),
        pltpu.VMEM((CHUNK, D), jnp.float32),
        pltpu.VMEM((ZROWS, D), jnp.float32),
    ] + [pltpu.SemaphoreType.DMA] * (3 * NBUF),
)


BLK = 1000


def _pre_body(x_ref, w_ref, b_ref, o_ref):
    o_ref[...] = jnp.maximum(
        jnp.dot(x_ref[...], w_ref[...], preferred_element_type=jnp.float32)
        + b_ref[...], 0.0)


_pre = pl.pallas_call(
    _pre_body,
    grid=(N // BLK,),
    in_specs=[pl.BlockSpec((BLK, D), lambda i: (i, 0)),
              pl.BlockSpec((D, D), lambda i: (0, 0)),
              pl.BlockSpec((1, D), lambda i: (0, 0))],
    out_specs=pl.BlockSpec((BLK, D), lambda i: (i, 0)),
    out_shape=jax.ShapeDtypeStruct((N, D), jnp.float32),
)


def _agg_from_parts(s_ref, d_ref):
    summed = s_ref[0] + s_ref[1]
    deg = d_ref[0, :, 0:1] + d_ref[1, :, 0:1]
    return summed * (1.0 / jnp.maximum(deg, 1.0))


def _combine_body(y_ref, s_ref, d_ref, ws_ref, wn_ref, b_ref, g_ref, be_ref,
                  o_ref):
    agg = _agg_from_parts(s_ref, d_ref)
    h = (jnp.dot(y_ref[...], ws_ref[...], preferred_element_type=jnp.float32)
         + jnp.dot(agg, wn_ref[...], preferred_element_type=jnp.float32)
         + b_ref[...])
    h = h * (_BN_SCALE * g_ref[...]) + be_ref[...]
    o_ref[...] = jnp.maximum(h, 0.0)


_combine = pl.pallas_call(
    _combine_body,
    grid=(N // BLK,),
    in_specs=[pl.BlockSpec((BLK, D), lambda i: (i, 0)),
              pl.BlockSpec((NC, BLK, D), lambda i: (0, i, 0)),
              pl.BlockSpec((NC, BLK, DEGW), lambda i: (0, i, 0)),
              pl.BlockSpec((D, D), lambda i: (0, 0)),
              pl.BlockSpec((D, D), lambda i: (0, 0)),
              pl.BlockSpec((1, D), lambda i: (0, 0)),
              pl.BlockSpec((1, D), lambda i: (0, 0)),
              pl.BlockSpec((1, D), lambda i: (0, 0))],
    out_specs=pl.BlockSpec((BLK, D), lambda i: (i, 0)),
    out_shape=jax.ShapeDtypeStruct((N, D), jnp.float32),
)


def _final_body(y_ref, s_ref, d_ref, ws_ref, wn_ref, b_ref, g_ref, be_ref,
                w2_ref, b2_ref, o_ref):
    agg = _agg_from_parts(s_ref, d_ref)
    h = (jnp.dot(y_ref[...], ws_ref[...], preferred_element_type=jnp.float32)
         + jnp.dot(agg, wn_ref[...], preferred_element_type=jnp.float32)
         + b_ref[...])
    h = h * (_BN_SCALE * g_ref[...]) + be_ref[...]
    o_ref[...] = (jnp.dot(h, w2_ref[...], preferred_element_type=jnp.float32)
                  + b2_ref[...])


_final = pl.pallas_call(
    _final_body,
    grid=(N // BLK,),
    in_specs=[pl.BlockSpec((BLK, D), lambda i: (i, 0)),
              pl.BlockSpec((NC, BLK, D), lambda i: (0, i, 0)),
              pl.BlockSpec((NC, BLK, DEGW), lambda i: (0, i, 0)),
              pl.BlockSpec((D, D), lambda i: (0, 0)),
              pl.BlockSpec((D, D), lambda i: (0, 0)),
              pl.BlockSpec((1, D), lambda i: (0, 0)),
              pl.BlockSpec((1, D), lambda i: (0, 0)),
              pl.BlockSpec((1, D), lambda i: (0, 0)),
              pl.BlockSpec((D, 1), lambda i: (0, 0)),
              pl.BlockSpec((1, 1), lambda i: (0, 0))],
    out_specs=pl.BlockSpec((BLK, 1), lambda i: (i, 0)),
    out_shape=jax.ShapeDtypeStruct((N, 1), jnp.float32),
)


def kernel(x, edge_index, W1, b1, Wself0, Wneigh0, bconv0, gamma0, beta0,
           Wself1, Wneigh1, bconv1, gamma1, beta1, W2, b2):
    zdeg = jnp.zeros((NPAD, DEGW), jnp.float32)
    ones = jnp.ones((CHUNK, DEGW), jnp.float32)

    (deg,) = _deg_pass(edge_index, zdeg, ones)
    y0 = _pre(x, W1, b1.reshape(1, D))
    (s0,) = _edge_pass(edge_index, y0)
    y1 = _combine(y0, s0, deg, Wself0, Wneigh0, bconv0.reshape(1, D),
                  gamma0.reshape(1, D), beta0.reshape(1, D))
    (s1,) = _edge_pass(edge_index, y1)
    out = _final(y1, s1, deg, Wself1, Wneigh1, bconv1.reshape(1, D),
                 gamma1.reshape(1, D), beta1.reshape(1, D), W2,
                 b2.reshape(1, 1))
    return out
